# baseline (speedup 1.0000x reference)
import functools

import jax
import jax.numpy as jnp
from jax import lax
from jax.experimental import pallas as pl
from jax.experimental.pallas import tpu as pltpu
from jax.experimental.pallas import tpu_sc as plsc

_N = 100000
_D = 128
_H = _D // 2
_L = 8
_NI = 8192

_NC = 2
_NS = 16
_NW = _NC * _NS
_CH = 128
_RG = 2 * _NI // _NW
_RS = _NI // _NW

_MESH = plsc.VectorSubcoreMesh(core_axis_name="c", subcore_axis_name="s")


def _wid():
    return lax.axis_index("s") * _NC + lax.axis_index("c")


@functools.partial(
    pl.kernel,
    out_type=jax.ShapeDtypeStruct((_L, _NI), jnp.int32),
    mesh=_MESH,
    scratch_types=[
        pltpu.VMEM((_NI,), jnp.int32),
        pltpu.VMEM((_N,), jnp.int32),
        pltpu.VMEM((_NI,), jnp.int32),
    ],
    compiler_params=pltpu.CompilerParams(needs_layout_passes=False),
)
def _winner_kernel(fathers_hbm, w_hbm, f_v, tab_v, w_v):
    wid = _wid()

    @pl.when(wid < _L)
    def _():
        pltpu.sync_copy(fathers_hbm.at[wid], f_v)

        def scat(j, carry):
            idx = f_v[pl.ds(j * 16, 16)]
            rows = lax.iota(jnp.int32, 16) + j * 16
            plsc.store_scatter(tab_v, [idx], rows)
            return carry

        lax.fori_loop(0, _NI // 16, scat, 0)

        def gath(j, carry):
            idx = f_v[pl.ds(j * 16, 16)]
            w_v[pl.ds(j * 16, 16)] = plsc.load_gather(tab_v, [idx])
            return carry

        lax.fori_loop(0, _NI // 16, gath, 0)
        pltpu.sync_copy(w_v, w_hbm.at[wid])


@functools.partial(
    pl.kernel,
    out_type=(
        jax.ShapeDtypeStruct((2 * _NI, _D), jnp.float32),
        jax.ShapeDtypeStruct((2 * _NI, _D), jnp.float32),
    ),
    mesh=_MESH,
    scratch_types=[
        pltpu.VMEM((_RG // _CH, _CH), jnp.int32),
        pltpu.VMEM((_RG, _D), jnp.float32),
        pltpu.VMEM((2, _CH, _D), jnp.float32),
        pltpu.SemaphoreType.DMA((_RG // _CH,)),
        pltpu.SemaphoreType.DMA((2,)),
        pltpu.SemaphoreType.DMA,
        pltpu.SemaphoreType.DMA((2,)),
    ],
)
def _gather_kernel(feat_hbm, xp_hbm, idx_hbm, pf_hbm, pp_hbm,
                   idx_v, f_v, p_v, semf, semp, semo, semox):
    wid = _wid()
    nch = _RG // _CH
    pltpu.sync_copy(idx_hbm.at[pl.ds(wid * nch, nch)], idx_v)
    gathers = [
        pltpu.async_copy(
            feat_hbm.at[idx_v.at[j]], f_v.at[pl.ds(j * _CH, _CH)],
            semf.at[j])
        for j in range(nch)
    ]
    xg = [
        pltpu.async_copy(xp_hbm.at[idx_v.at[j]], p_v.at[j], semp.at[j])
        for j in range(2)
    ]
    outs = []
    for j in range(nch):
        gathers[j].wait()
        outs.append(pltpu.async_copy(
            f_v.at[pl.ds(j * _CH, _CH)],
            pf_hbm.at[pl.ds(wid * _RG + j * _CH, _CH)], semo))
    xouts = [None, None]
    for j in range(nch):
        b = j % 2
        xg[b].wait()
        xouts[b] = pltpu.async_copy(
            p_v.at[b],
            pp_hbm.at[pl.ds(wid * _RG + j * _CH, _CH)], semox.at[b])
        if j + 2 < nch:
            xouts[b].wait()
            xouts[b] = None
            xg[b] = pltpu.async_copy(
                xp_hbm.at[idx_v.at[j + 2]], p_v.at[b], semp.at[b])
    for c in outs:
        c.wait()
    for c in xouts:
        if c is not None:
            c.wait()


@functools.partial(
    pl.kernel,
    out_type=(),
    mesh=_MESH,
    scratch_types=[
        pltpu.VMEM((_RS // _CH, _CH), jnp.int32),
        pltpu.VMEM((_RS // _CH, _CH), jnp.int32),
        pltpu.VMEM((_RS, _D), jnp.float32),
        pltpu.SemaphoreType.DMA((_RS // _CH,)),
        pltpu.SemaphoreType.DMA,
    ],
)
def _scatter_kernel(out_hbm, w_hbm, fa_hbm, feat_hbm, w_v, fa_v, buf_v,
                    semg, sems):
    wid = _wid()
    nch = _RS // _CH
    pltpu.sync_copy(w_hbm.at[pl.ds(wid * nch, nch)], w_v)
    pltpu.sync_copy(fa_hbm.at[pl.ds(wid * nch, nch)], fa_v)
    gathers = [
        pltpu.async_copy(out_hbm.at[w_v.at[j]],
                         buf_v.at[pl.ds(j * _CH, _CH)], semg.at[j])
        for j in range(nch)
    ]
    outs = []
    for j in range(nch):
        gathers[j].wait()
        outs.append(pltpu.async_copy(
            buf_v.at[pl.ds(j * _CH, _CH)], feat_hbm.at[fa_v.at[j]], sems))
    for c in outs:
        c.wait()


_XB = 2000


def _pad_body(x_ref, o_ref):
    o_ref[...] = jnp.pad(x_ref[...], ((0, 0), (0, _D - 5)))


_pad_x = pl.pallas_call(
    _pad_body,
    grid=(_N // _XB,),
    in_specs=[pl.BlockSpec((_XB, 5), lambda j: (j, 0))],
    out_specs=pl.BlockSpec((_XB, _D), lambda j: (j, 0)),
    out_shape=jax.ShapeDtypeStruct((_N, _D), jnp.float32),
)


_BR = 2048


def _lstm_body(lp_ref, rp_ref, lf_ref, rf_ref, wih_ref, whh_ref,
               bih_ref, bhh_ref, out_ref):
    wih = wih_ref[...]
    whh = whh_ref[...]
    b = bih_ref[...] + bhh_ref[...]

    def sig(x):
        return 0.5 * jnp.tanh(0.5 * x) + 0.5

    def cell(p, f):
        h = f[:, :_H]
        c = f[:, _H:]
        gates = (jnp.dot(p, wih, preferred_element_type=jnp.float32)
                 + jnp.dot(h, whh, preferred_element_type=jnp.float32) + b)
        i = sig(gates[:, 0:_H])
        fg = sig(gates[:, _H:2 * _H])
        g = jnp.tanh(gates[:, 2 * _H:3 * _H])
        o = sig(gates[:, 3 * _H:4 * _H])
        c2 = fg * c + i * g
        h2 = o * jnp.tanh(c2)
        return h2, c2

    hl, cl = cell(lp_ref[:, :16], lf_ref[...])
    hr, cr = cell(rp_ref[:, :16], rf_ref[...])
    out_ref[...] = jnp.concatenate([hl + hr, cl + cr], axis=1)


_lstm = pl.pallas_call(
    _lstm_body,
    grid=(_NI // _BR,),
    in_specs=[
        pl.BlockSpec((_BR, _D), lambda j: (j, 0)),
        pl.BlockSpec((_BR, _D), lambda j: (j + _NI // _BR, 0)),
        pl.BlockSpec((_BR, _D), lambda j: (j, 0)),
        pl.BlockSpec((_BR, _D), lambda j: (j + _NI // _BR, 0)),
        pl.BlockSpec((16, 4 * _H), lambda j: (0, 0)),
        pl.BlockSpec((_H, 4 * _H), lambda j: (0, 0)),
        pl.BlockSpec((1, 4 * _H), lambda j: (0, 0)),
        pl.BlockSpec((1, 4 * _H), lambda j: (0, 0)),
    ],
    out_specs=pl.BlockSpec((_BR, _D), lambda j: (j, 0)),
    out_shape=jax.ShapeDtypeStruct((_NI, _D), jnp.float32),
)


def kernel(X, Feature, I_list, W_ih, W_hh, b_ih, b_hh):
    it = jnp.transpose(I_list[:, 0], (0, 2, 1))
    fathers = it[:, 2]
    pairs_idx = jnp.concatenate([it[:, 0], it[:, 1]], axis=1)
    pairs2d = pairs_idx.reshape(_L, 2 * _NI // _CH, _CH)
    fa2d = fathers.reshape(_L, _NI // _CH, _CH)
    xp = _pad_x(X)
    wih_t = jnp.pad(W_ih, ((0, 0), (0, 11))).T
    whh_t = W_hh.T
    bih2 = b_ih.reshape(1, 4 * _H)
    bhh2 = b_hh.reshape(1, 4 * _H)

    w_all = _winner_kernel(fathers)
    w2d = w_all.reshape(_L, _NI // _CH, _CH)

    pf, pp = _gather_kernel(Feature, xp, pairs2d[0])
    out = _lstm(pp, pp, pf, pf, wih_t, whh_t, bih2, bhh2)
    feat_ref = jax.new_ref(Feature)
    _scatter_kernel(out, w2d[0], fa2d[0], feat_ref)
    for l in range(1, _L):
        pf, pp = _gather_kernel(feat_ref, xp, pairs2d[l])
        out = _lstm(pp, pp, pf, pf, wih_t, whh_t, bih2, bhh2)
        _scatter_kernel(out, w2d[l], fa2d[l], feat_ref)
    return jax.freeze(feat_ref)


# speedup vs baseline: 1.0459x; 1.0459x over previous
import functools

import jax
import jax.numpy as jnp
from jax import lax
from jax.experimental import pallas as pl
from jax.experimental.pallas import tpu as pltpu
from jax.experimental.pallas import tpu_sc as plsc

_N = 100000
_D = 128
_H = _D // 2
_L = 8
_NI = 8192

_NC = 2
_NS = 16
_NW = _NC * _NS
_CH = 128
_RG = 2 * _NI // _NW
_RS = _NI // _NW

_MESH = plsc.VectorSubcoreMesh(core_axis_name="c", subcore_axis_name="s")


def _wid():
    return lax.axis_index("s") * _NC + lax.axis_index("c")


@functools.partial(
    pl.kernel,
    out_type=jax.ShapeDtypeStruct((_L, _NI), jnp.int32),
    mesh=_MESH,
    scratch_types=[
        pltpu.VMEM((_NI,), jnp.int32),
        pltpu.VMEM((_N,), jnp.int32),
        pltpu.VMEM((_NI,), jnp.int32),
    ],
    compiler_params=pltpu.CompilerParams(needs_layout_passes=False),
)
def _winner_kernel(fathers_hbm, w_hbm, f_v, tab_v, w_v):
    wid = _wid()

    @pl.when(wid < _L)
    def _():
        pltpu.sync_copy(fathers_hbm.at[wid], f_v)

        def scat(j, carry):
            idx = f_v[pl.ds(j * 16, 16)]
            rows = lax.iota(jnp.int32, 16) + j * 16
            plsc.store_scatter(tab_v, [idx], rows)
            return carry

        lax.fori_loop(0, _NI // 16, scat, 0)

        def gath(j, carry):
            idx = f_v[pl.ds(j * 16, 16)]
            w_v[pl.ds(j * 16, 16)] = plsc.load_gather(tab_v, [idx])
            return carry

        lax.fori_loop(0, _NI // 16, gath, 0)
        pltpu.sync_copy(w_v, w_hbm.at[wid])


@functools.partial(
    pl.kernel,
    out_type=(
        jax.ShapeDtypeStruct((2 * _NI, _D), jnp.float32),
        jax.ShapeDtypeStruct((2 * _NI, _D), jnp.float32),
    ),
    mesh=_MESH,
    scratch_types=[
        pltpu.VMEM((_RG // _CH, _CH), jnp.int32),
        pltpu.VMEM((_RG, _D), jnp.float32),
        pltpu.VMEM((2, _CH, _D), jnp.float32),
        pltpu.SemaphoreType.DMA((_RG // _CH,)),
        pltpu.SemaphoreType.DMA((2,)),
        pltpu.SemaphoreType.DMA,
        pltpu.SemaphoreType.DMA((2,)),
    ],
)
def _gather_kernel(feat_hbm, xp_hbm, idx_hbm, pf_hbm, pp_hbm,
                   idx_v, f_v, p_v, semf, semp, semo, semox):
    wid = _wid()
    nch = _RG // _CH
    pltpu.sync_copy(idx_hbm.at[pl.ds(wid * nch, nch)], idx_v)
    gathers = [
        pltpu.async_copy(
            feat_hbm.at[idx_v.at[j]], f_v.at[pl.ds(j * _CH, _CH)],
            semf.at[j])
        for j in range(nch)
    ]
    xg = [
        pltpu.async_copy(xp_hbm.at[idx_v.at[j]], p_v.at[j], semp.at[j])
        for j in range(2)
    ]
    outs = []
    for j in range(nch):
        gathers[j].wait()
        outs.append(pltpu.async_copy(
            f_v.at[pl.ds(j * _CH, _CH)],
            pf_hbm.at[pl.ds(wid * _RG + j * _CH, _CH)], semo))
    xouts = [None, None]
    for j in range(nch):
        b = j % 2
        xg[b].wait()
        xouts[b] = pltpu.async_copy(
            p_v.at[b],
            pp_hbm.at[pl.ds(wid * _RG + j * _CH, _CH)], semox.at[b])
        if j + 2 < nch:
            xouts[b].wait()
            xouts[b] = None
            xg[b] = pltpu.async_copy(
                xp_hbm.at[idx_v.at[j + 2]], p_v.at[b], semp.at[b])
    for c in outs:
        c.wait()
    for c in xouts:
        if c is not None:
            c.wait()


@functools.partial(
    pl.kernel,
    out_type=(),
    mesh=_MESH,
    scratch_types=[
        pltpu.VMEM((_RS // _CH, _CH), jnp.int32),
        pltpu.VMEM((_RS // _CH, _CH), jnp.int32),
        pltpu.VMEM((_RS, _D), jnp.float32),
        pltpu.SemaphoreType.DMA((_RS // _CH,)),
        pltpu.SemaphoreType.DMA,
    ],
)
def _scatter_kernel(out_hbm, w_hbm, fa_hbm, feat_hbm, w_v, fa_v, buf_v,
                    semg, sems):
    wid = _wid()
    nch = _RS // _CH
    pltpu.sync_copy(w_hbm.at[pl.ds(wid * nch, nch)], w_v)
    pltpu.sync_copy(fa_hbm.at[pl.ds(wid * nch, nch)], fa_v)
    gathers = [
        pltpu.async_copy(out_hbm.at[w_v.at[j]],
                         buf_v.at[pl.ds(j * _CH, _CH)], semg.at[j])
        for j in range(nch)
    ]
    outs = []
    for j in range(nch):
        gathers[j].wait()
        outs.append(pltpu.async_copy(
            buf_v.at[pl.ds(j * _CH, _CH)], feat_hbm.at[fa_v.at[j]], sems))
    for c in outs:
        c.wait()


_BR = 2048


def _lstm_body(lp_ref, rp_ref, lf_ref, rf_ref, wih_ref, whh_ref,
               bih_ref, bhh_ref, out_ref):
    wih = wih_ref[...]
    whh = whh_ref[...]
    b = bih_ref[...] + bhh_ref[...]

    def sig(x):
        return 0.5 * jnp.tanh(0.5 * x) + 0.5

    def cell(p, f):
        h = f[:, :_H]
        c = f[:, _H:]
        gates = (jnp.dot(p, wih, preferred_element_type=jnp.float32)
                 + jnp.dot(h, whh, preferred_element_type=jnp.float32) + b)
        i = sig(gates[:, 0:_H])
        fg = sig(gates[:, _H:2 * _H])
        g = jnp.tanh(gates[:, 2 * _H:3 * _H])
        o = sig(gates[:, 3 * _H:4 * _H])
        c2 = fg * c + i * g
        h2 = o * jnp.tanh(c2)
        return h2, c2

    hl, cl = cell(lp_ref[:, :16], lf_ref[...])
    hr, cr = cell(rp_ref[:, :16], rf_ref[...])
    out_ref[...] = jnp.concatenate([hl + hr, cl + cr], axis=1)


_lstm = pl.pallas_call(
    _lstm_body,
    grid=(_NI // _BR,),
    in_specs=[
        pl.BlockSpec((_BR, _D), lambda j: (j, 0)),
        pl.BlockSpec((_BR, _D), lambda j: (j + _NI // _BR, 0)),
        pl.BlockSpec((_BR, _D), lambda j: (j, 0)),
        pl.BlockSpec((_BR, _D), lambda j: (j + _NI // _BR, 0)),
        pl.BlockSpec((16, 4 * _H), lambda j: (0, 0)),
        pl.BlockSpec((_H, 4 * _H), lambda j: (0, 0)),
        pl.BlockSpec((1, 4 * _H), lambda j: (0, 0)),
        pl.BlockSpec((1, 4 * _H), lambda j: (0, 0)),
    ],
    out_specs=pl.BlockSpec((_BR, _D), lambda j: (j, 0)),
    out_shape=jax.ShapeDtypeStruct((_NI, _D), jnp.float32),
)


def kernel(X, Feature, I_list, W_ih, W_hh, b_ih, b_hh):
    it = jnp.transpose(I_list[:, 0], (0, 2, 1))
    fathers = it[:, 2]
    pairs_idx = jnp.concatenate([it[:, 0], it[:, 1]], axis=1)
    pairs2d = pairs_idx.reshape(_L, 2 * _NI // _CH, _CH)
    fa2d = fathers.reshape(_L, _NI // _CH, _CH)
    xp = jnp.concatenate(
        [X, jnp.zeros((_N, _D - 5), jnp.float32)], axis=1)
    wih_t = jnp.pad(W_ih, ((0, 0), (0, 11))).T
    whh_t = W_hh.T
    bih2 = b_ih.reshape(1, 4 * _H)
    bhh2 = b_hh.reshape(1, 4 * _H)

    w_all = _winner_kernel(fathers)
    w2d = w_all.reshape(_L, _NI // _CH, _CH)

    pf, pp = _gather_kernel(Feature, xp, pairs2d[0])
    out = _lstm(pp, pp, pf, pf, wih_t, whh_t, bih2, bhh2)
    feat_ref = jax.new_ref(Feature)
    _scatter_kernel(out, w2d[0], fa2d[0], feat_ref)
    for l in range(1, _L):
        pf, pp = _gather_kernel(feat_ref, xp, pairs2d[l])
        out = _lstm(pp, pp, pf, pf, wih_t, whh_t, bih2, bhh2)
        _scatter_kernel(out, w2d[l], fa2d[l], feat_ref)
    return jax.freeze(feat_ref)


# speedup vs baseline: 1.0487x; 1.0026x over previous
import functools

import jax
import jax.numpy as jnp
from jax import lax
from jax.experimental import pallas as pl
from jax.experimental.pallas import tpu as pltpu
from jax.experimental.pallas import tpu_sc as plsc

_N = 100000
_D = 128
_H = _D // 2
_L = 8
_NI = 8192

_NC = 2
_NS = 16
_NW = _NC * _NS
_CH = 128
_RG = 2 * _NI // _NW
_RS = _NI // _NW

_MESH = plsc.VectorSubcoreMesh(core_axis_name="c", subcore_axis_name="s")


def _wid():
    return lax.axis_index("s") * _NC + lax.axis_index("c")


@functools.partial(
    pl.kernel,
    out_type=jax.ShapeDtypeStruct((_L, _NI), jnp.int32),
    mesh=_MESH,
    scratch_types=[
        pltpu.VMEM((_NI,), jnp.int32),
        pltpu.VMEM((_N,), jnp.int32),
        pltpu.VMEM((_NI,), jnp.int32),
    ],
    compiler_params=pltpu.CompilerParams(needs_layout_passes=False),
)
def _winner_kernel(fathers_hbm, w_hbm, f_v, tab_v, w_v):
    wid = _wid()

    @pl.when(wid < _L)
    def _():
        pltpu.sync_copy(fathers_hbm.at[wid], f_v)

        def scat(j, carry):
            idx = f_v[pl.ds(j * 16, 16)]
            rows = lax.iota(jnp.int32, 16) + j * 16
            plsc.store_scatter(tab_v, [idx], rows)
            return carry

        lax.fori_loop(0, _NI // 16, scat, 0)

        def gath(j, carry):
            idx = f_v[pl.ds(j * 16, 16)]
            w_v[pl.ds(j * 16, 16)] = plsc.load_gather(tab_v, [idx])
            return carry

        lax.fori_loop(0, _NI // 16, gath, 0)
        pltpu.sync_copy(w_v, w_hbm.at[wid])


@functools.partial(
    pl.kernel,
    out_type=(
        jax.ShapeDtypeStruct((2 * _NI, _D), jnp.float32),
        jax.ShapeDtypeStruct((2 * _NI, _D), jnp.float32),
    ),
    mesh=_MESH,
    scratch_types=[
        pltpu.VMEM((_RG // _CH, _CH), jnp.int32),
        pltpu.VMEM((_RG, _D), jnp.float32),
        pltpu.VMEM((2, _CH, _D), jnp.float32),
        pltpu.SemaphoreType.DMA((_RG // _CH,)),
        pltpu.SemaphoreType.DMA((2,)),
        pltpu.SemaphoreType.DMA,
        pltpu.SemaphoreType.DMA((2,)),
    ],
)
def _gather_kernel(feat_hbm, xp_hbm, idx_hbm, pf_hbm, pp_hbm,
                   idx_v, f_v, p_v, semf, semp, semo, semox):
    wid = _wid()
    nch = _RG // _CH
    pltpu.sync_copy(idx_hbm.at[pl.ds(wid * nch, nch)], idx_v)
    gathers = [
        pltpu.async_copy(
            feat_hbm.at[idx_v.at[j]], f_v.at[pl.ds(j * _CH, _CH)],
            semf.at[j])
        for j in range(nch)
    ]
    xg = [
        pltpu.async_copy(xp_hbm.at[idx_v.at[j]], p_v.at[j], semp.at[j])
        for j in range(2)
    ]
    outs = []
    for j in range(nch):
        gathers[j].wait()
        outs.append(pltpu.async_copy(
            f_v.at[pl.ds(j * _CH, _CH)],
            pf_hbm.at[pl.ds(wid * _RG + j * _CH, _CH)], semo))
    xouts = [None, None]
    for j in range(nch):
        b = j % 2
        xg[b].wait()
        xouts[b] = pltpu.async_copy(
            p_v.at[b],
            pp_hbm.at[pl.ds(wid * _RG + j * _CH, _CH)], semox.at[b])
        if j + 2 < nch:
            xouts[b].wait()
            xouts[b] = None
            xg[b] = pltpu.async_copy(
                xp_hbm.at[idx_v.at[j + 2]], p_v.at[b], semp.at[b])
    for c in outs:
        c.wait()
    for c in xouts:
        if c is not None:
            c.wait()


@functools.partial(
    pl.kernel,
    out_type=(),
    mesh=_MESH,
    scratch_types=[
        pltpu.VMEM((_RS // _CH, _CH), jnp.int32),
        pltpu.VMEM((_RS // _CH, _CH), jnp.int32),
        pltpu.VMEM((_RS, _D), jnp.float32),
        pltpu.SemaphoreType.DMA((_RS // _CH,)),
        pltpu.SemaphoreType.DMA,
    ],
)
def _scatter_kernel(out_hbm, w_hbm, fa_hbm, feat_hbm, w_v, fa_v, buf_v,
                    semg, sems):
    wid = _wid()
    nch = _RS // _CH
    pltpu.sync_copy(w_hbm.at[pl.ds(wid * nch, nch)], w_v)
    pltpu.sync_copy(fa_hbm.at[pl.ds(wid * nch, nch)], fa_v)
    gathers = [
        pltpu.async_copy(out_hbm.at[w_v.at[j]],
                         buf_v.at[pl.ds(j * _CH, _CH)], semg.at[j])
        for j in range(nch)
    ]
    outs = []
    for j in range(nch):
        gathers[j].wait()
        outs.append(pltpu.async_copy(
            buf_v.at[pl.ds(j * _CH, _CH)], feat_hbm.at[fa_v.at[j]], sems))
    for c in outs:
        c.wait()


_BR = 2048


def _lstm_body(lp_ref, rp_ref, lf_ref, rf_ref, wih_ref, whh_ref,
               bih_ref, bhh_ref, out_ref):
    wih = wih_ref[...]
    whh = whh_ref[...]
    b = bih_ref[...] + bhh_ref[...]

    def sig(x):
        return 0.5 * jnp.tanh(0.5 * x) + 0.5

    def cell(p, f):
        h = f[:, :_H]
        c = f[:, _H:]
        gates = (jnp.dot(p, wih, preferred_element_type=jnp.float32)
                 + jnp.dot(h, whh, preferred_element_type=jnp.float32) + b)
        i = sig(gates[:, 0:_H])
        fg = sig(gates[:, _H:2 * _H])
        g = jnp.tanh(gates[:, 2 * _H:3 * _H])
        o = sig(gates[:, 3 * _H:4 * _H])
        c2 = fg * c + i * g
        h2 = o * jnp.tanh(c2)
        return h2, c2

    hl, cl = cell(lp_ref[:, :16], lf_ref[...])
    hr, cr = cell(rp_ref[:, :16], rf_ref[...])
    out_ref[...] = jnp.concatenate([hl + hr, cl + cr], axis=1)


_lstm = pl.pallas_call(
    _lstm_body,
    grid=(_NI // _BR,),
    in_specs=[
        pl.BlockSpec((_BR, _D), lambda j: (j, 0)),
        pl.BlockSpec((_BR, _D), lambda j: (j + _NI // _BR, 0)),
        pl.BlockSpec((_BR, _D), lambda j: (j, 0)),
        pl.BlockSpec((_BR, _D), lambda j: (j + _NI // _BR, 0)),
        pl.BlockSpec((16, 4 * _H), lambda j: (0, 0)),
        pl.BlockSpec((_H, 4 * _H), lambda j: (0, 0)),
        pl.BlockSpec((1, 4 * _H), lambda j: (0, 0)),
        pl.BlockSpec((1, 4 * _H), lambda j: (0, 0)),
    ],
    out_specs=pl.BlockSpec((_BR, _D), lambda j: (j, 0)),
    out_shape=jax.ShapeDtypeStruct((_NI, _D), jnp.float32),
)


def kernel(X, Feature, I_list, W_ih, W_hh, b_ih, b_hh):
    it = jnp.transpose(I_list[:, 0], (0, 2, 1))
    fathers = it[:, 2]
    pairs_idx = jnp.concatenate([it[:, 0], it[:, 1]], axis=1)
    pairs2d = pairs_idx.reshape(_L, 2 * _NI // _CH, _CH)
    fa2d = fathers.reshape(_L, _NI // _CH, _CH)
    xp = jnp.concatenate(
        [X, jnp.zeros((_N, _D - 5), jnp.float32)], axis=1)
    wih_t = jnp.pad(W_ih, ((0, 0), (0, 11))).T
    whh_t = W_hh.T
    bih2 = b_ih.reshape(1, 4 * _H)
    bhh2 = b_hh.reshape(1, 4 * _H)

    w_all = _winner_kernel(fathers)
    w2d = w_all.reshape(_L, _NI // _CH, _CH)

    pf, pp = _gather_kernel(Feature, xp, pairs2d[0])
    out = _lstm(pp, pp, pf, pf, wih_t, whh_t, bih2, bhh2)
    feat_src, _ = jax.lax.optimization_barrier((Feature, xp))
    feat_ref = jax.new_ref(feat_src)
    _scatter_kernel(out, w2d[0], fa2d[0], feat_ref)
    for l in range(1, _L):
        pf, pp = _gather_kernel(feat_ref, xp, pairs2d[l])
        out = _lstm(pp, pp, pf, pf, wih_t, whh_t, bih2, bhh2)
        _scatter_kernel(out, w2d[l], fa2d[l], feat_ref)
    return jax.freeze(feat_ref)
